# Initial kernel scaffold; baseline (speedup 1.0000x reference)
import jax
import jax.numpy as jnp
from jax.experimental import pallas as pl


def kernel(questions, e_s, subj, rel_idx, obj, params):
    raise NotImplementedError("write your pallas kernel here")


# speedup vs baseline: 5.4272x; 5.4272x over previous
import functools

import jax
import jax.numpy as jnp
from jax import lax
from jax.experimental import pallas as pl
from jax.experimental.pallas import tpu as pltpu
from jax.experimental.pallas import tpu_sc as plsc

BSZ = 32
L = 16
NUM_ENT = 50000
NUM_REL = 512
DIM_H = 512
STEPS = 3
T_TRIPLES = 500000

NC = 2
NS = 16
LN = 16

EP = 51200
EPT = EP // NS
RSUB = 640
NCH = EPT // RSUB
C = 1024
PT = 31744
TP = PT * NS
RELP = NUM_REL + 1


def _run_gru(x, W_ih, W_hh, b_ih, b_hh, reverse):
    B = x.shape[0]
    Hh = W_hh.shape[1]
    xs = jnp.swapaxes(x, 0, 1)
    if reverse:
        xs = xs[::-1]

    def step(h, xt):
        gx = xt @ W_ih.T + b_ih
        gh = h @ W_hh.T + b_hh
        xr, xz, xn = jnp.split(gx, 3, axis=-1)
        hr, hz, hn = jnp.split(gh, 3, axis=-1)
        r = jax.nn.sigmoid(xr + hr)
        z = jax.nn.sigmoid(xz + hz)
        n = jnp.tanh(xn + r * hn)
        hnew = (1.0 - z) * n + z * h
        return hnew, hnew

    h0 = jnp.zeros((B, Hh), dtype=x.dtype)
    _, hs = jax.lax.scan(step, h0, xs)
    if reverse:
        hs = hs[::-1]
    return jnp.swapaxes(hs, 0, 1)


def _dense_encoder(questions, params):
    question_lens = L - jnp.sum((questions == 0).astype(jnp.int32), axis=1)
    q_word_emb = params['word_emb'][questions]
    h_f = _run_gru(q_word_emb, params['gru_f_Wih'], params['gru_f_Whh'],
                   params['gru_f_bih'], params['gru_f_bhh'], False)
    h_b = _run_gru(q_word_emb, params['gru_b_Wih'], params['gru_b_Whh'],
                   params['gru_b_bih'], params['gru_b_bhh'], True)
    q_word_h = jnp.concatenate([h_f, h_b], axis=-1)

    att_mask = (jnp.arange(L)[None, :] < question_lens[:, None])

    q_word_h_hop = q_word_h
    prev_dist_ctx = None
    rel_dists = []
    ctx_h_list = []
    for t in range(STEPS):
        h_key = q_word_h_hop @ params['W_step'][t] + params['b_step'][t]
        q_logits = h_key @ jnp.swapaxes(q_word_h, -1, -2)
        q_logits = jnp.swapaxes(q_logits, -1, -2)
        q_dist = jax.nn.softmax(q_logits, axis=2)
        q_dist = q_dist * att_mask.astype(jnp.float32)[:, None, :]
        q_dist = q_dist / (jnp.sum(q_dist, axis=2, keepdims=True) + 1e-06)
        hop_ctx = q_dist @ q_word_h_hop
        if t == 0:
            q_word_h_hop = q_word_h + hop_ctx
            dist_ctx = hop_ctx
        else:
            z = jax.nn.sigmoid(prev_dist_ctx @ params['hw_W'] + params['hw_b'])
            q_word_h_hop = q_word_h + hop_ctx + z * prev_dist_ctx
            dist_ctx = hop_ctx + z * prev_dist_ctx
        prev_dist_ctx = dist_ctx
        q_word_att = jnp.sum(q_dist, axis=1, keepdims=True)
        q_word_att = jax.nn.softmax(q_word_att, axis=2)
        q_word_att = q_word_att / (jnp.sum(q_word_att, axis=2, keepdims=True) + 1e-06)
        ctx_h = jnp.squeeze(jnp.swapaxes(q_word_h_hop, -1, -2) @ jnp.swapaxes(q_word_att, -1, -2), axis=2)
        ctx_h_list.append(ctx_h)
        rel_dists.append(jax.nn.sigmoid(ctx_h @ params['rel_W'] + params['rel_b']))

    gates = []
    for t in range(1, STEPS):
        prev_rel = jnp.argmax(rel_dists[t - 1], axis=1)
        curr_rel = jnp.argmax(rel_dists[t], axis=1)
        cond = (jnp.abs(prev_rel - curr_rel) == 1) & (jnp.remainder(jnp.minimum(prev_rel, curr_rel), 2) == 0)
        gates.append(cond.astype(jnp.float32))

    ctx_hist = jnp.stack(ctx_h_list, axis=2)
    hop_logit = jnp.swapaxes(ctx_hist, -1, -2) @ params['hop_w'] + params['hop_b']
    hop_attn = jnp.swapaxes(jax.nn.softmax(jnp.swapaxes(hop_logit, -1, -2), axis=2), -1, -2)
    hop_att_tmp = jnp.squeeze(hop_attn, axis=2)
    loc = jnp.argmax(hop_att_tmp, axis=-1)
    loc_gt1 = (loc > 1).astype(jnp.float32)
    return rel_dists, gates, hop_att_tmp, loc_gt1


def _pack_rows(x):
    n = x.shape[1]
    return jnp.transpose(x.reshape(NC, LN, n), (0, 2, 1))


def _sc_follow(e_pair, rel0, rel1, rel2, subj_p, relx_p, obj_p, scal):
    f32 = jnp.float32
    mesh = plsc.VectorSubcoreMesh(core_axis_name="c", subcore_axis_name="s")
    out_sd = jax.ShapeDtypeStruct((NC * EP, LN), f32)

    @functools.partial(
        pl.kernel, mesh=mesh,
        out_type=[out_sd, out_sd, out_sd],
        compiler_params=pltpu.CompilerParams(use_tc_tiling_on_sc=False),
        scratch_types=[
            pltpu.VMEM((C,), jnp.int32),
            pltpu.VMEM((C,), jnp.int32),
            pltpu.VMEM((C,), jnp.int32),
            pltpu.VMEM((C, LN), f32),
            pltpu.VMEM((C, LN), f32),
            pltpu.VMEM((RSUB, LN), f32),
            pltpu.VMEM((RSUB, LN), f32),
            pltpu.VMEM((RSUB, LN), f32),
            pltpu.VMEM((RSUB, LN), f32),
            pltpu.VMEM((8, LN), f32),
            pltpu.VMEM_SHARED((EP, LN), f32),
            pltpu.SemaphoreType.DMA,
        ],
    )
    def k(e_hbm, r0_hbm, r1_hbm, r2_hbm, subj_hbm, relx_hbm, obj_hbm, scal_hbm,
          ent0, ent1, outp,
          subj_v, rel_v, obj_v, erow, rrow, stage, prev0, prev1, zbuf, scal_v,
          acc, sem):
        c = lax.axis_index("c")
        s = lax.axis_index("s")
        zero16 = jnp.zeros((LN,), f32)

        def zb(i, carry):
            zbuf[i] = zero16
            return carry
        lax.fori_loop(0, RSUB, zb, 0)

        pltpu.sync_copy(scal_hbm.at[c], scal_v)

        ebase = s * EPT
        for kk in range(NCH):
            pltpu.sync_copy(zbuf, acc.at[pl.ds(ebase + kk * RSUB, RSUB)])
        plsc.subcore_barrier()

        gate1 = scal_v[0]
        gate2 = scal_v[1]
        a0 = scal_v[2]
        a1 = scal_v[3]
        a2 = scal_v[4]
        locgt1 = scal_v[5]

        eoff = lax.broadcast(c * EP, (LN,))
        roff = lax.broadcast(c * RELP, (LN,))
        tri_base = s * PT

        for t in range(STEPS):
            src = [e_hbm, ent0, ent1][t]
            rel_hbm = [r0_hbm, r1_hbm, r2_hbm][t]
            dst = [ent0, ent1, outp][t]

            def chunk_body(j, carry):
                b = tri_base + j * C
                pltpu.sync_copy(subj_hbm.at[pl.ds(b, C)], subj_v)
                pltpu.sync_copy(relx_hbm.at[pl.ds(b, C)], rel_v)
                pltpu.sync_copy(obj_hbm.at[pl.ds(b, C)], obj_v)

                def adj(i, cy):
                    o = i * LN
                    subj_v[pl.ds(o, LN)] = subj_v[pl.ds(o, LN)] + eoff
                    rel_v[pl.ds(o, LN)] = rel_v[pl.ds(o, LN)] + roff
                    return cy
                lax.fori_loop(0, C // LN, adj, 0)

                cp1 = pltpu.async_copy(src.at[subj_v], erow, sem)
                cp2 = pltpu.async_copy(rel_hbm.at[rel_v], rrow, sem)
                cp1.wait()
                cp2.wait()

                def mul(i, cy):
                    erow[i] = erow[i] * rrow[i]
                    return cy
                lax.fori_loop(0, C, mul, 0)

                pltpu.sync_copy(erow, acc.at[obj_v], add=True)
                return carry
            lax.fori_loop(0, PT // C, chunk_body, 0)
            plsc.subcore_barrier()

            for kk in range(NCH):
                rb = ebase + kk * RSUB
                gb = c * EP + rb
                pltpu.sync_copy(acc.at[pl.ds(rb, RSUB)], stage)
                if t == 1:
                    pltpu.sync_copy(e_hbm.at[pl.ds(gb, RSUB)], prev0)
                elif t == 2:
                    pltpu.sync_copy(ent0.at[pl.ds(gb, RSUB)], prev0)
                    pltpu.sync_copy(ent1.at[pl.ds(gb, RSUB)], prev1)

                if t == 0:
                    def norm(i, cy):
                        stage[i] = jnp.minimum(stage[i], 1.0)
                        return cy
                elif t == 1:
                    def norm(i, cy):
                        v = jnp.minimum(stage[i], 1.0)
                        g = jnp.where(prev0[i] > 0.9, gate1, 0.0)
                        stage[i] = v * (1.0 - g)
                        return cy
                else:
                    def norm(i, cy):
                        v = jnp.minimum(stage[i], 1.0)
                        e0 = prev0[i]
                        g = jnp.where(e0 > 0.9, gate2, 0.0)
                        v = v * (1.0 - g)
                        comb = a0 * e0 + a1 * prev1[i] + a2 * v
                        w0 = jnp.where(e0 > 0.0, locgt1, 0.0)
                        stage[i] = comb * (1.0 - 0.1 * w0)
                        return cy
                lax.fori_loop(0, RSUB, norm, 0)

                pltpu.sync_copy(stage, dst.at[pl.ds(gb, RSUB)])
                if t < STEPS - 1:
                    pltpu.sync_copy(zbuf, acc.at[pl.ds(rb, RSUB)])
            plsc.subcore_barrier()

    return k(e_pair, rel0, rel1, rel2, subj_p, relx_p, obj_p, scal)


def kernel(questions, e_s, subj, rel_idx, obj, params):
    rel_dists, gates, hop_att, loc_gt1 = _dense_encoder(questions, params)

    e_pair = _pack_rows(e_s)
    e_pair = jnp.pad(e_pair, ((0, 0), (0, EP - NUM_ENT), (0, 0)))
    e_pair = e_pair.reshape(NC * EP, LN)

    rels = []
    for t in range(STEPS):
        r = _pack_rows(rel_dists[t])
        r = jnp.pad(r, ((0, 0), (0, 1), (0, 0)))
        rels.append(r.reshape(NC * RELP, LN))

    pad_n = TP - T_TRIPLES
    subj_p = jnp.concatenate([subj, jnp.zeros((pad_n,), jnp.int32)])
    relx_p = jnp.concatenate([rel_idx, jnp.full((pad_n,), NUM_REL, jnp.int32)])
    obj_p = jnp.concatenate([obj, jnp.zeros((pad_n,), jnp.int32)])

    scal6 = jnp.stack([gates[0], gates[1], hop_att[:, 0], hop_att[:, 1],
                       hop_att[:, 2], loc_gt1])
    scal = jnp.transpose(scal6.reshape(6, NC, LN), (1, 0, 2))
    scal = jnp.pad(scal, ((0, 0), (0, 2), (0, 0)))

    _, _, outp = _sc_follow(e_pair, rels[0], rels[1], rels[2],
                            subj_p, relx_p, obj_p, scal)

    out = outp.reshape(NC, EP, LN)[:, :NUM_ENT, :]
    return jnp.transpose(out, (0, 2, 1)).reshape(BSZ, NUM_ENT)


# speedup vs baseline: 7.2130x; 1.3290x over previous
import functools

import jax
import jax.numpy as jnp
from jax import lax
from jax.experimental import pallas as pl
from jax.experimental.pallas import tpu as pltpu
from jax.experimental.pallas import tpu_sc as plsc

BSZ = 32
L = 16
NUM_ENT = 50000
NUM_REL = 512
DIM_H = 512
STEPS = 3
T_TRIPLES = 500000

NC = 2
NS = 16
LN = 16

EP = 51200
EPT = EP // NS
RSUB = 640
NCH = EPT // RSUB
C = 1024
PT = 31744
TP = PT * NS
RELP = NUM_REL + 1


def _run_gru(x, W_ih, W_hh, b_ih, b_hh, reverse):
    B = x.shape[0]
    Hh = W_hh.shape[1]
    xs = jnp.swapaxes(x, 0, 1)
    if reverse:
        xs = xs[::-1]

    def step(h, xt):
        gx = xt @ W_ih.T + b_ih
        gh = h @ W_hh.T + b_hh
        xr, xz, xn = jnp.split(gx, 3, axis=-1)
        hr, hz, hn = jnp.split(gh, 3, axis=-1)
        r = jax.nn.sigmoid(xr + hr)
        z = jax.nn.sigmoid(xz + hz)
        n = jnp.tanh(xn + r * hn)
        hnew = (1.0 - z) * n + z * h
        return hnew, hnew

    h0 = jnp.zeros((B, Hh), dtype=x.dtype)
    _, hs = jax.lax.scan(step, h0, xs)
    if reverse:
        hs = hs[::-1]
    return jnp.swapaxes(hs, 0, 1)


def _dense_encoder(questions, params):
    question_lens = L - jnp.sum((questions == 0).astype(jnp.int32), axis=1)
    q_word_emb = params['word_emb'][questions]
    h_f = _run_gru(q_word_emb, params['gru_f_Wih'], params['gru_f_Whh'],
                   params['gru_f_bih'], params['gru_f_bhh'], False)
    h_b = _run_gru(q_word_emb, params['gru_b_Wih'], params['gru_b_Whh'],
                   params['gru_b_bih'], params['gru_b_bhh'], True)
    q_word_h = jnp.concatenate([h_f, h_b], axis=-1)

    att_mask = (jnp.arange(L)[None, :] < question_lens[:, None])

    q_word_h_hop = q_word_h
    prev_dist_ctx = None
    rel_dists = []
    ctx_h_list = []
    for t in range(STEPS):
        h_key = q_word_h_hop @ params['W_step'][t] + params['b_step'][t]
        q_logits = h_key @ jnp.swapaxes(q_word_h, -1, -2)
        q_logits = jnp.swapaxes(q_logits, -1, -2)
        q_dist = jax.nn.softmax(q_logits, axis=2)
        q_dist = q_dist * att_mask.astype(jnp.float32)[:, None, :]
        q_dist = q_dist / (jnp.sum(q_dist, axis=2, keepdims=True) + 1e-06)
        hop_ctx = q_dist @ q_word_h_hop
        if t == 0:
            q_word_h_hop = q_word_h + hop_ctx
            dist_ctx = hop_ctx
        else:
            z = jax.nn.sigmoid(prev_dist_ctx @ params['hw_W'] + params['hw_b'])
            q_word_h_hop = q_word_h + hop_ctx + z * prev_dist_ctx
            dist_ctx = hop_ctx + z * prev_dist_ctx
        prev_dist_ctx = dist_ctx
        q_word_att = jnp.sum(q_dist, axis=1, keepdims=True)
        q_word_att = jax.nn.softmax(q_word_att, axis=2)
        q_word_att = q_word_att / (jnp.sum(q_word_att, axis=2, keepdims=True) + 1e-06)
        ctx_h = jnp.squeeze(jnp.swapaxes(q_word_h_hop, -1, -2) @ jnp.swapaxes(q_word_att, -1, -2), axis=2)
        ctx_h_list.append(ctx_h)
        rel_dists.append(jax.nn.sigmoid(ctx_h @ params['rel_W'] + params['rel_b']))

    gates = []
    for t in range(1, STEPS):
        prev_rel = jnp.argmax(rel_dists[t - 1], axis=1)
        curr_rel = jnp.argmax(rel_dists[t], axis=1)
        cond = (jnp.abs(prev_rel - curr_rel) == 1) & (jnp.remainder(jnp.minimum(prev_rel, curr_rel), 2) == 0)
        gates.append(cond.astype(jnp.float32))

    ctx_hist = jnp.stack(ctx_h_list, axis=2)
    hop_logit = jnp.swapaxes(ctx_hist, -1, -2) @ params['hop_w'] + params['hop_b']
    hop_attn = jnp.swapaxes(jax.nn.softmax(jnp.swapaxes(hop_logit, -1, -2), axis=2), -1, -2)
    hop_att_tmp = jnp.squeeze(hop_attn, axis=2)
    loc = jnp.argmax(hop_att_tmp, axis=-1)
    loc_gt1 = (loc > 1).astype(jnp.float32)
    return rel_dists, gates, hop_att_tmp, loc_gt1


def _pack_rows(x):
    n = x.shape[1]
    return jnp.transpose(x.reshape(NC, LN, n), (0, 2, 1))


def _sc_follow(e_pair, rel0, rel1, rel2, subj_p, relx_p, obj_p, scal):
    f32 = jnp.float32
    mesh = plsc.VectorSubcoreMesh(core_axis_name="c", subcore_axis_name="s")
    out_sd = jax.ShapeDtypeStruct((NC * EP, LN), f32)

    @functools.partial(
        pl.kernel, mesh=mesh,
        out_type=[out_sd, out_sd, out_sd],
        compiler_params=pltpu.CompilerParams(use_tc_tiling_on_sc=False),
        scratch_types=[
            pltpu.VMEM((C,), jnp.int32),
            pltpu.VMEM((C,), jnp.int32),
            pltpu.VMEM((C,), jnp.int32),
            pltpu.VMEM((C, LN), f32),
            pltpu.VMEM((C, LN), f32),
            pltpu.VMEM((RSUB, LN), f32),
            pltpu.VMEM((RSUB, LN), f32),
            pltpu.VMEM((RSUB, LN), f32),
            pltpu.VMEM((RSUB, LN), f32),
            pltpu.VMEM((8, LN), f32),
            pltpu.VMEM_SHARED((EP, LN), f32),
            pltpu.SemaphoreType.DMA,
        ],
    )
    def k(e_hbm, r0_hbm, r1_hbm, r2_hbm, subj_hbm, relx_hbm, obj_hbm, scal_hbm,
          ent0, ent1, outp,
          subj_v, rel_v, obj_v, erow, rrow, stage, prev0, prev1, zbuf, scal_v,
          acc, sem):
        c = lax.axis_index("c")
        s = lax.axis_index("s")
        zero16 = jnp.zeros((LN,), f32)

        @plsc.parallel_loop(0, RSUB, unroll=8)
        def _(i):
            zbuf[i] = zero16

        pltpu.sync_copy(scal_hbm.at[c], scal_v)

        ebase = s * EPT
        for kk in range(NCH):
            pltpu.sync_copy(zbuf, acc.at[pl.ds(ebase + kk * RSUB, RSUB)])
        plsc.subcore_barrier()

        gate1 = scal_v[0]
        gate2 = scal_v[1]
        a0 = scal_v[2]
        a1 = scal_v[3]
        a2 = scal_v[4]
        locgt1 = scal_v[5]

        eoff = lax.broadcast(c * EP, (LN,))
        roff = lax.broadcast(c * RELP, (LN,))
        tri_base = s * PT

        for t in range(STEPS):
            src = [e_hbm, ent0, ent1][t]
            rel_hbm = [r0_hbm, r1_hbm, r2_hbm][t]
            dst = [ent0, ent1, outp][t]

            def chunk_body(j, carry):
                b = tri_base + j * C
                pltpu.sync_copy(subj_hbm.at[pl.ds(b, C)], subj_v)
                pltpu.sync_copy(relx_hbm.at[pl.ds(b, C)], rel_v)
                pltpu.sync_copy(obj_hbm.at[pl.ds(b, C)], obj_v)

                @plsc.parallel_loop(0, C, step=LN, unroll=4)
                def _(o):
                    subj_v[pl.ds(o, LN)] = subj_v[pl.ds(o, LN)] + eoff
                    rel_v[pl.ds(o, LN)] = rel_v[pl.ds(o, LN)] + roff

                cp1 = pltpu.async_copy(src.at[subj_v], erow, sem)
                cp2 = pltpu.async_copy(rel_hbm.at[rel_v], rrow, sem)
                cp1.wait()
                cp2.wait()

                @plsc.parallel_loop(0, C, unroll=8)
                def _(i):
                    erow[i] = erow[i] * rrow[i]

                pltpu.sync_copy(erow, acc.at[obj_v], add=True)
                return carry
            lax.fori_loop(0, PT // C, chunk_body, 0)
            plsc.subcore_barrier()

            for kk in range(NCH):
                rb = ebase + kk * RSUB
                gb = c * EP + rb
                pltpu.sync_copy(acc.at[pl.ds(rb, RSUB)], stage)
                if t == 1:
                    pltpu.sync_copy(e_hbm.at[pl.ds(gb, RSUB)], prev0)
                elif t == 2:
                    pltpu.sync_copy(ent0.at[pl.ds(gb, RSUB)], prev0)
                    pltpu.sync_copy(ent1.at[pl.ds(gb, RSUB)], prev1)

                if t == 0:
                    def norm(i):
                        stage[i] = jnp.minimum(stage[i], 1.0)
                elif t == 1:
                    def norm(i):
                        v = jnp.minimum(stage[i], 1.0)
                        g = jnp.where(prev0[i] > 0.9, gate1, 0.0)
                        stage[i] = v * (1.0 - g)
                else:
                    def norm(i):
                        v = jnp.minimum(stage[i], 1.0)
                        e0 = prev0[i]
                        g = jnp.where(e0 > 0.9, gate2, 0.0)
                        v = v * (1.0 - g)
                        comb = a0 * e0 + a1 * prev1[i] + a2 * v
                        w0 = jnp.where(e0 > 0.0, locgt1, 0.0)
                        stage[i] = comb * (1.0 - 0.1 * w0)
                plsc.parallel_loop(0, RSUB, unroll=8)(norm)

                pltpu.sync_copy(stage, dst.at[pl.ds(gb, RSUB)])
                if t < STEPS - 1:
                    pltpu.sync_copy(zbuf, acc.at[pl.ds(rb, RSUB)])
            plsc.subcore_barrier()

    return k(e_pair, rel0, rel1, rel2, subj_p, relx_p, obj_p, scal)


def kernel(questions, e_s, subj, rel_idx, obj, params):
    rel_dists, gates, hop_att, loc_gt1 = _dense_encoder(questions, params)

    e_pair = _pack_rows(e_s)
    e_pair = jnp.pad(e_pair, ((0, 0), (0, EP - NUM_ENT), (0, 0)))
    e_pair = e_pair.reshape(NC * EP, LN)

    rels = []
    for t in range(STEPS):
        r = _pack_rows(rel_dists[t])
        r = jnp.pad(r, ((0, 0), (0, 1), (0, 0)))
        rels.append(r.reshape(NC * RELP, LN))

    pad_n = TP - T_TRIPLES
    subj_p = jnp.concatenate([subj, jnp.zeros((pad_n,), jnp.int32)])
    relx_p = jnp.concatenate([rel_idx, jnp.full((pad_n,), NUM_REL, jnp.int32)])
    obj_p = jnp.concatenate([obj, jnp.zeros((pad_n,), jnp.int32)])

    scal6 = jnp.stack([gates[0], gates[1], hop_att[:, 0], hop_att[:, 1],
                       hop_att[:, 2], loc_gt1])
    scal = jnp.transpose(scal6.reshape(6, NC, LN), (1, 0, 2))
    scal = jnp.pad(scal, ((0, 0), (0, 2), (0, 0)))

    _, _, outp = _sc_follow(e_pair, rels[0], rels[1], rels[2],
                            subj_p, relx_p, obj_p, scal)

    out = outp.reshape(NC, EP, LN)[:, :NUM_ENT, :]
    return jnp.transpose(out, (0, 2, 1)).reshape(BSZ, NUM_ENT)


# speedup vs baseline: 8.4094x; 1.1659x over previous
import functools

import jax
import jax.numpy as jnp
from jax import lax
from jax.experimental import pallas as pl
from jax.experimental.pallas import tpu as pltpu
from jax.experimental.pallas import tpu_sc as plsc

BSZ = 32
L = 16
NUM_ENT = 50000
NUM_REL = 512
DIM_H = 512
STEPS = 3
T_TRIPLES = 500000

NC = 2
NS = 16
LN = 16

EP = 51200
EPT = EP // NS
RSUB = 640
NCH = EPT // RSUB
C = 512
PT = 31744
TP = PT * NS
RELP = NUM_REL + 1


def _run_gru(x, W_ih, W_hh, b_ih, b_hh, reverse):
    B = x.shape[0]
    Hh = W_hh.shape[1]
    xs = jnp.swapaxes(x, 0, 1)
    if reverse:
        xs = xs[::-1]

    def step(h, xt):
        gx = xt @ W_ih.T + b_ih
        gh = h @ W_hh.T + b_hh
        xr, xz, xn = jnp.split(gx, 3, axis=-1)
        hr, hz, hn = jnp.split(gh, 3, axis=-1)
        r = jax.nn.sigmoid(xr + hr)
        z = jax.nn.sigmoid(xz + hz)
        n = jnp.tanh(xn + r * hn)
        hnew = (1.0 - z) * n + z * h
        return hnew, hnew

    h0 = jnp.zeros((B, Hh), dtype=x.dtype)
    _, hs = jax.lax.scan(step, h0, xs)
    if reverse:
        hs = hs[::-1]
    return jnp.swapaxes(hs, 0, 1)


def _dense_encoder(questions, params):
    question_lens = L - jnp.sum((questions == 0).astype(jnp.int32), axis=1)
    q_word_emb = params['word_emb'][questions]
    h_f = _run_gru(q_word_emb, params['gru_f_Wih'], params['gru_f_Whh'],
                   params['gru_f_bih'], params['gru_f_bhh'], False)
    h_b = _run_gru(q_word_emb, params['gru_b_Wih'], params['gru_b_Whh'],
                   params['gru_b_bih'], params['gru_b_bhh'], True)
    q_word_h = jnp.concatenate([h_f, h_b], axis=-1)

    att_mask = (jnp.arange(L)[None, :] < question_lens[:, None])

    q_word_h_hop = q_word_h
    prev_dist_ctx = None
    rel_dists = []
    ctx_h_list = []
    for t in range(STEPS):
        h_key = q_word_h_hop @ params['W_step'][t] + params['b_step'][t]
        q_logits = h_key @ jnp.swapaxes(q_word_h, -1, -2)
        q_logits = jnp.swapaxes(q_logits, -1, -2)
        q_dist = jax.nn.softmax(q_logits, axis=2)
        q_dist = q_dist * att_mask.astype(jnp.float32)[:, None, :]
        q_dist = q_dist / (jnp.sum(q_dist, axis=2, keepdims=True) + 1e-06)
        hop_ctx = q_dist @ q_word_h_hop
        if t == 0:
            q_word_h_hop = q_word_h + hop_ctx
            dist_ctx = hop_ctx
        else:
            z = jax.nn.sigmoid(prev_dist_ctx @ params['hw_W'] + params['hw_b'])
            q_word_h_hop = q_word_h + hop_ctx + z * prev_dist_ctx
            dist_ctx = hop_ctx + z * prev_dist_ctx
        prev_dist_ctx = dist_ctx
        q_word_att = jnp.sum(q_dist, axis=1, keepdims=True)
        q_word_att = jax.nn.softmax(q_word_att, axis=2)
        q_word_att = q_word_att / (jnp.sum(q_word_att, axis=2, keepdims=True) + 1e-06)
        ctx_h = jnp.squeeze(jnp.swapaxes(q_word_h_hop, -1, -2) @ jnp.swapaxes(q_word_att, -1, -2), axis=2)
        ctx_h_list.append(ctx_h)
        rel_dists.append(jax.nn.sigmoid(ctx_h @ params['rel_W'] + params['rel_b']))

    gates = []
    for t in range(1, STEPS):
        prev_rel = jnp.argmax(rel_dists[t - 1], axis=1)
        curr_rel = jnp.argmax(rel_dists[t], axis=1)
        cond = (jnp.abs(prev_rel - curr_rel) == 1) & (jnp.remainder(jnp.minimum(prev_rel, curr_rel), 2) == 0)
        gates.append(cond.astype(jnp.float32))

    ctx_hist = jnp.stack(ctx_h_list, axis=2)
    hop_logit = jnp.swapaxes(ctx_hist, -1, -2) @ params['hop_w'] + params['hop_b']
    hop_attn = jnp.swapaxes(jax.nn.softmax(jnp.swapaxes(hop_logit, -1, -2), axis=2), -1, -2)
    hop_att_tmp = jnp.squeeze(hop_attn, axis=2)
    loc = jnp.argmax(hop_att_tmp, axis=-1)
    loc_gt1 = (loc > 1).astype(jnp.float32)
    return rel_dists, gates, hop_att_tmp, loc_gt1


def _pack_rows(x):
    n = x.shape[1]
    return jnp.transpose(x.reshape(NC, LN, n), (0, 2, 1))


def _sc_follow(e_pair, rel0, rel1, rel2, subj_p, relx_p, obj_p, scal):
    f32 = jnp.float32
    mesh = plsc.VectorSubcoreMesh(core_axis_name="c", subcore_axis_name="s")
    out_sd = jax.ShapeDtypeStruct((NC * EP, LN), f32)

    @functools.partial(
        pl.kernel, mesh=mesh,
        out_type=[out_sd, out_sd, out_sd],
        compiler_params=pltpu.CompilerParams(use_tc_tiling_on_sc=False),
        scratch_types=[
            [pltpu.VMEM((C,), jnp.int32)] * 2,
            [pltpu.VMEM((C,), jnp.int32)] * 2,
            [pltpu.VMEM((C,), jnp.int32)] * 2,
            [pltpu.VMEM((C, LN), f32)] * 2,
            [pltpu.VMEM((C, LN), f32)] * 2,
            pltpu.VMEM((RSUB, LN), f32),
            pltpu.VMEM((RSUB, LN), f32),
            pltpu.VMEM((RSUB, LN), f32),
            pltpu.VMEM((RSUB, LN), f32),
            pltpu.VMEM((8, LN), f32),
            pltpu.VMEM_SHARED((EP, LN), f32),
            [pltpu.SemaphoreType.DMA] * 2,
            [pltpu.SemaphoreType.DMA] * 2,
        ],
    )
    def k(e_hbm, r0_hbm, r1_hbm, r2_hbm, subj_hbm, relx_hbm, obj_hbm, scal_hbm,
          ent0, ent1, outp,
          subj_v, rel_v, obj_v, erow, rrow, stage, prev0, prev1, zbuf, scal_v,
          acc, sem_g, sem_s):
        c = lax.axis_index("c")
        s = lax.axis_index("s")
        zero16 = jnp.zeros((LN,), f32)

        @plsc.parallel_loop(0, RSUB, unroll=8)
        def _(i):
            zbuf[i] = zero16

        pltpu.sync_copy(scal_hbm.at[c], scal_v)

        ebase = s * EPT
        for kk in range(NCH):
            pltpu.sync_copy(zbuf, acc.at[pl.ds(ebase + kk * RSUB, RSUB)])
        plsc.subcore_barrier()

        gate1 = scal_v[0]
        gate2 = scal_v[1]
        a0 = scal_v[2]
        a1 = scal_v[3]
        a2 = scal_v[4]
        locgt1 = scal_v[5]

        eoff = lax.broadcast(c * EP, (LN,))
        roff = lax.broadcast(c * RELP, (LN,))
        tri_base = s * PT

        NCHK = PT // C

        for t in range(STEPS):
            src = [e_hbm, ent0, ent1][t]
            rel_hbm = [r0_hbm, r1_hbm, r2_hbm][t]
            dst = [ent0, ent1, outp][t]

            def phase1(j, b):
                @pl.when(j >= 2)
                def _():
                    pltpu.make_async_copy(erow[b], acc.at[obj_v[b]], sem_s[b]).wait()
                tb = tri_base + j * C
                pltpu.sync_copy(subj_hbm.at[pl.ds(tb, C)], subj_v[b])
                pltpu.sync_copy(relx_hbm.at[pl.ds(tb, C)], rel_v[b])
                pltpu.sync_copy(obj_hbm.at[pl.ds(tb, C)], obj_v[b])

                @plsc.parallel_loop(0, C, step=LN, unroll=4)
                def _(o):
                    subj_v[b][pl.ds(o, LN)] = subj_v[b][pl.ds(o, LN)] + eoff
                    rel_v[b][pl.ds(o, LN)] = rel_v[b][pl.ds(o, LN)] + roff

                pltpu.async_copy(src.at[subj_v[b]], erow[b], sem_g[b])
                pltpu.async_copy(rel_hbm.at[rel_v[b]], rrow[b], sem_g[b])

            def phase2(j, b):
                pltpu.make_async_copy(src.at[subj_v[b]], erow[b], sem_g[b]).wait()
                pltpu.make_async_copy(rel_hbm.at[rel_v[b]], rrow[b], sem_g[b]).wait()

                @plsc.parallel_loop(0, C, unroll=8)
                def _(i):
                    erow[b][i] = erow[b][i] * rrow[b][i]

                pltpu.async_copy(erow[b], acc.at[obj_v[b]], sem_s[b], add=True)

            phase1(0, 0)

            def pipe_body(it, carry):
                j0 = it * 2
                phase1(j0 + 1, 1)
                phase2(j0, 0)
                phase2(j0 + 1, 1)

                @pl.when(j0 + 2 < NCHK)
                def _():
                    phase1(j0 + 2, 0)
                return carry
            lax.fori_loop(0, NCHK // 2, pipe_body, 0)

            pltpu.make_async_copy(erow[0], acc.at[obj_v[0]], sem_s[0]).wait()
            pltpu.make_async_copy(erow[1], acc.at[obj_v[1]], sem_s[1]).wait()
            plsc.subcore_barrier()

            for kk in range(NCH):
                rb = ebase + kk * RSUB
                gb = c * EP + rb
                pltpu.sync_copy(acc.at[pl.ds(rb, RSUB)], stage)
                if t == 1:
                    pltpu.sync_copy(e_hbm.at[pl.ds(gb, RSUB)], prev0)
                elif t == 2:
                    pltpu.sync_copy(ent0.at[pl.ds(gb, RSUB)], prev0)
                    pltpu.sync_copy(ent1.at[pl.ds(gb, RSUB)], prev1)

                if t == 0:
                    def norm(i):
                        stage[i] = jnp.minimum(stage[i], 1.0)
                elif t == 1:
                    def norm(i):
                        v = jnp.minimum(stage[i], 1.0)
                        g = jnp.where(prev0[i] > 0.9, gate1, 0.0)
                        stage[i] = v * (1.0 - g)
                else:
                    def norm(i):
                        v = jnp.minimum(stage[i], 1.0)
                        e0 = prev0[i]
                        g = jnp.where(e0 > 0.9, gate2, 0.0)
                        v = v * (1.0 - g)
                        comb = a0 * e0 + a1 * prev1[i] + a2 * v
                        w0 = jnp.where(e0 > 0.0, locgt1, 0.0)
                        stage[i] = comb * (1.0 - 0.1 * w0)
                plsc.parallel_loop(0, RSUB, unroll=8)(norm)

                pltpu.sync_copy(stage, dst.at[pl.ds(gb, RSUB)])
                if t < STEPS - 1:
                    pltpu.sync_copy(zbuf, acc.at[pl.ds(rb, RSUB)])
            plsc.subcore_barrier()

    return k(e_pair, rel0, rel1, rel2, subj_p, relx_p, obj_p, scal)


def kernel(questions, e_s, subj, rel_idx, obj, params):
    rel_dists, gates, hop_att, loc_gt1 = _dense_encoder(questions, params)

    e_pair = _pack_rows(e_s)
    e_pair = jnp.pad(e_pair, ((0, 0), (0, EP - NUM_ENT), (0, 0)))
    e_pair = e_pair.reshape(NC * EP, LN)

    rels = []
    for t in range(STEPS):
        r = _pack_rows(rel_dists[t])
        r = jnp.pad(r, ((0, 0), (0, 1), (0, 0)))
        rels.append(r.reshape(NC * RELP, LN))

    pad_n = TP - T_TRIPLES
    subj_p = jnp.concatenate([subj, jnp.zeros((pad_n,), jnp.int32)])
    relx_p = jnp.concatenate([rel_idx, jnp.full((pad_n,), NUM_REL, jnp.int32)])
    obj_p = jnp.concatenate([obj, jnp.zeros((pad_n,), jnp.int32)])

    scal6 = jnp.stack([gates[0], gates[1], hop_att[:, 0], hop_att[:, 1],
                       hop_att[:, 2], loc_gt1])
    scal = jnp.transpose(scal6.reshape(6, NC, LN), (1, 0, 2))
    scal = jnp.pad(scal, ((0, 0), (0, 2), (0, 0)))

    _, _, outp = _sc_follow(e_pair, rels[0], rels[1], rels[2],
                            subj_p, relx_p, obj_p, scal)

    out = outp.reshape(NC, EP, LN)[:, :NUM_ENT, :]
    return jnp.transpose(out, (0, 2, 1)).reshape(BSZ, NUM_ENT)
